# baseline (speedup 1.0000x reference)
import jax
import jax.numpy as jnp
from jax.experimental import pallas as pl
from jax.experimental.pallas import tpu as pltpu

B, D_IN, D_H, D_OUT = 64, 2048, 2048, 2048
K_MEM, D_RET = 100, 5
K_BLK = 256
N1 = D_IN // K_BLK


def _body(x_ref, tgt_ref, w1_ref, b1_ref, lnw_ref, lnb_ref, w2_ref, b2_ref,
          out_ref, scal_ref, acc_ref, feats_ref):
    i = pl.program_id(0)

    @pl.when(i == 0)
    def _init():
        acc_ref[...] = jnp.zeros_like(acc_ref)

    @pl.when(i < N1)
    def _mm1():
        acc_ref[...] += jnp.dot(x_ref[...], w1_ref[...],
                                preferred_element_type=jnp.float32)

    @pl.when(i == N1 - 1)
    def _ln():
        h = acc_ref[...] + b1_ref[...]
        mu = jnp.mean(h, axis=-1, keepdims=True)
        var = jnp.mean((h - mu) ** 2, axis=-1, keepdims=True)
        ln = (h - mu) / jnp.sqrt(var + 1e-5) * lnw_ref[...] + lnb_ref[...]
        feats = jnp.maximum(ln, 0.0)
        for j in range(N1):
            feats_ref[j] = feats[:, j * K_BLK:(j + 1) * K_BLK]
        acc_ref[...] = jnp.zeros_like(acc_ref)

    @pl.when(i >= N1)
    def _mm2():
        j = i - N1
        acc_ref[...] += jnp.dot(feats_ref[j], w2_ref[...],
                                preferred_element_type=jnp.float32)

    @pl.when(i == N1)
    def _retrieve():
        feats = jnp.concatenate([feats_ref[j] for j in range(N1)], axis=1)
        keys = jnp.mean(feats, axis=0, keepdims=True)
        keys_n = jnp.maximum(jnp.sqrt(jnp.sum(keys * keys)), 1e-8)
        rn = jnp.sqrt(jnp.sum(feats * feats, axis=1, keepdims=True))
        dots = jnp.sum(feats * keys, axis=1, keepdims=True)
        distances = dots / (jnp.maximum(rn, 1e-8) * keys_n)

        iota = jax.lax.broadcasted_iota(jnp.int32, (B, 1), 0)
        work = distances
        sel = jnp.zeros((B, 1), dtype=jnp.float32)
        for _ in range(D_RET):
            m = jnp.min(work)
            first = jnp.min(jnp.where(work == m, iota, B))
            pick = iota == first
            sel = jnp.where(pick, 1.0, sel)
            work = jnp.where(pick, 99.0, work)

        support_mean = jnp.sum(feats * sel, axis=0, keepdims=True) / D_RET
        sm_n = jnp.maximum(jnp.sqrt(jnp.sum(support_mean * support_mean)),
                           1e-12)
        centers = support_mean / sm_n
        feats_n = jnp.mean(feats / jnp.maximum(rn, 1e-12), axis=0,
                           keepdims=True)
        fn_n = jnp.maximum(jnp.sqrt(jnp.sum(feats_n * feats_n)), 1e-8)
        c_n = jnp.maximum(jnp.sqrt(jnp.sum(centers * centers)), 1e-8)
        cos = jnp.sum(feats_n * centers) / (fn_n * c_n)
        dist = 1.0 - cos
        adjusted_lr = 2e-05 * jnp.exp(-dist * 0.01)

        lane = jax.lax.broadcasted_iota(jnp.int32, (1, 128), 1)
        scal_ref[...] = jnp.where(lane == 1, dist,
                                  jnp.where(lane == 2, adjusted_lr, 0.0))

    @pl.when(i == 2 * N1 - 1)
    def _final():
        out = acc_ref[...] + b2_ref[...]
        out_ref[...] = out
        t = tgt_ref[...]
        d = out - t
        sq_mean = jnp.mean(d * d)
        rmse = jnp.sqrt(sq_mean)
        nmse = sq_mean / jnp.mean(t * t)
        loss = rmse + nmse
        lane = jax.lax.broadcasted_iota(jnp.int32, (1, 128), 1)
        scal_ref[...] = jnp.where(lane == 0, loss, scal_ref[...])


def kernel(inputs, target, W1, b1, ln_w, ln_b, W2, b2):
    grid = (2 * N1,)
    out, scal = pl.pallas_call(
        _body,
        grid=grid,
        in_specs=[
            pl.BlockSpec((B, K_BLK), lambda i: (0, jnp.minimum(i, N1 - 1))),
            pl.BlockSpec((B, D_OUT), lambda i: (0, 0)),
            pl.BlockSpec((K_BLK, D_H), lambda i: (jnp.minimum(i, N1 - 1), 0)),
            pl.BlockSpec((1, D_H), lambda i: (0, 0)),
            pl.BlockSpec((1, D_H), lambda i: (0, 0)),
            pl.BlockSpec((1, D_H), lambda i: (0, 0)),
            pl.BlockSpec((K_BLK, D_OUT), lambda i: (jnp.maximum(i - N1, 0), 0)),
            pl.BlockSpec((1, D_OUT), lambda i: (0, 0)),
        ],
        out_specs=[
            pl.BlockSpec((B, D_OUT), lambda i: (0, 0)),
            pl.BlockSpec((1, 128), lambda i: (0, 0)),
        ],
        out_shape=[
            jax.ShapeDtypeStruct((B, D_OUT), jnp.float32),
            jax.ShapeDtypeStruct((1, 128), jnp.float32),
        ],
        scratch_shapes=[
            pltpu.VMEM((B, D_H), jnp.float32),
            pltpu.VMEM((N1, B, K_BLK), jnp.float32),
        ],
        compiler_params=pltpu.CompilerParams(
            dimension_semantics=("arbitrary",),
        ),
    )(
        inputs, target, W1,
        b1.reshape(1, D_H), ln_w.reshape(1, D_H), ln_b.reshape(1, D_H),
        W2, b2.reshape(1, D_OUT),
    )
    return (out, scal[0, 0], scal[0, 1], scal[0, 2])


# speedup vs baseline: 1.2408x; 1.2408x over previous
import jax
import jax.numpy as jnp
from jax.experimental import pallas as pl
from jax.experimental.pallas import tpu as pltpu

B, D_IN, D_H, D_OUT = 64, 2048, 2048, 2048
K_MEM, D_RET = 100, 5
K_BLK = 1024
N1 = D_IN // K_BLK


def _body(x_ref, tgt_ref, w1_ref, b1_ref, lnw_ref, lnb_ref, w2_ref, b2_ref,
          out_ref, scal_ref, acc_ref, feats_ref):
    i = pl.program_id(0)

    @pl.when(i == 0)
    def _init():
        acc_ref[...] = jnp.zeros_like(acc_ref)

    @pl.when(i < N1)
    def _mm1():
        acc_ref[...] += jnp.dot(x_ref[...], w1_ref[...],
                                preferred_element_type=jnp.float32)

    @pl.when(i == N1 - 1)
    def _ln():
        h = acc_ref[...] + b1_ref[...]
        mu = jnp.mean(h, axis=-1, keepdims=True)
        var = jnp.mean((h - mu) ** 2, axis=-1, keepdims=True)
        ln = (h - mu) / jnp.sqrt(var + 1e-5) * lnw_ref[...] + lnb_ref[...]
        feats = jnp.maximum(ln, 0.0)
        for j in range(N1):
            feats_ref[j] = feats[:, j * K_BLK:(j + 1) * K_BLK]
        acc_ref[...] = jnp.zeros_like(acc_ref)

    @pl.when(i >= N1)
    def _mm2():
        j = i - N1
        acc_ref[...] += jnp.dot(feats_ref[j], w2_ref[...],
                                preferred_element_type=jnp.float32)

    @pl.when(i == N1)
    def _retrieve():
        feats = jnp.concatenate([feats_ref[j] for j in range(N1)], axis=1)
        keys = jnp.mean(feats, axis=0, keepdims=True)
        keys_n = jnp.maximum(jnp.sqrt(jnp.sum(keys * keys)), 1e-8)
        rn = jnp.sqrt(jnp.sum(feats * feats, axis=1, keepdims=True))
        dots = jnp.sum(feats * keys, axis=1, keepdims=True)
        distances = dots / (jnp.maximum(rn, 1e-8) * keys_n)

        iota = jax.lax.broadcasted_iota(jnp.int32, (B, 1), 0)
        work = distances
        sel = jnp.zeros((B, 1), dtype=jnp.float32)
        for _ in range(D_RET):
            m = jnp.min(work)
            first = jnp.min(jnp.where(work == m, iota, B))
            pick = iota == first
            sel = jnp.where(pick, 1.0, sel)
            work = jnp.where(pick, 99.0, work)

        support_mean = jnp.sum(feats * sel, axis=0, keepdims=True) / D_RET
        sm_n = jnp.maximum(jnp.sqrt(jnp.sum(support_mean * support_mean)),
                           1e-12)
        centers = support_mean / sm_n
        feats_n = jnp.mean(feats / jnp.maximum(rn, 1e-12), axis=0,
                           keepdims=True)
        fn_n = jnp.maximum(jnp.sqrt(jnp.sum(feats_n * feats_n)), 1e-8)
        c_n = jnp.maximum(jnp.sqrt(jnp.sum(centers * centers)), 1e-8)
        cos = jnp.sum(feats_n * centers) / (fn_n * c_n)
        dist = 1.0 - cos
        adjusted_lr = 2e-05 * jnp.exp(-dist * 0.01)

        lane = jax.lax.broadcasted_iota(jnp.int32, (1, 128), 1)
        scal_ref[...] = jnp.where(lane == 1, dist,
                                  jnp.where(lane == 2, adjusted_lr, 0.0))

    @pl.when(i == 2 * N1 - 1)
    def _final():
        out = acc_ref[...] + b2_ref[...]
        out_ref[...] = out
        t = tgt_ref[...]
        d = out - t
        sq_mean = jnp.mean(d * d)
        rmse = jnp.sqrt(sq_mean)
        nmse = sq_mean / jnp.mean(t * t)
        loss = rmse + nmse
        lane = jax.lax.broadcasted_iota(jnp.int32, (1, 128), 1)
        scal_ref[...] = jnp.where(lane == 0, loss, scal_ref[...])


def kernel(inputs, target, W1, b1, ln_w, ln_b, W2, b2):
    grid = (2 * N1,)
    out, scal = pl.pallas_call(
        _body,
        grid=grid,
        in_specs=[
            pl.BlockSpec((B, K_BLK), lambda i: (0, jnp.minimum(i, N1 - 1))),
            pl.BlockSpec((B, D_OUT), lambda i: (0, 0)),
            pl.BlockSpec((K_BLK, D_H), lambda i: (jnp.minimum(i, N1 - 1), 0)),
            pl.BlockSpec((1, D_H), lambda i: (0, 0)),
            pl.BlockSpec((1, D_H), lambda i: (0, 0)),
            pl.BlockSpec((1, D_H), lambda i: (0, 0)),
            pl.BlockSpec((K_BLK, D_OUT), lambda i: (jnp.maximum(i - N1, 0), 0)),
            pl.BlockSpec((1, D_OUT), lambda i: (0, 0)),
        ],
        out_specs=[
            pl.BlockSpec((B, D_OUT), lambda i: (0, 0)),
            pl.BlockSpec((1, 128), lambda i: (0, 0)),
        ],
        out_shape=[
            jax.ShapeDtypeStruct((B, D_OUT), jnp.float32),
            jax.ShapeDtypeStruct((1, 128), jnp.float32),
        ],
        scratch_shapes=[
            pltpu.VMEM((B, D_H), jnp.float32),
            pltpu.VMEM((N1, B, K_BLK), jnp.float32),
        ],
        compiler_params=pltpu.CompilerParams(
            dimension_semantics=("arbitrary",),
        ),
    )(
        inputs, target, W1,
        b1.reshape(1, D_H), ln_w.reshape(1, D_H), ln_b.reshape(1, D_H),
        W2, b2.reshape(1, D_OUT),
    )
    return (out, scal[0, 0], scal[0, 1], scal[0, 2])
